# baseline (speedup 1.0000x reference)
import functools

import jax
import jax.numpy as jnp
from jax.experimental import pallas as pl
from jax.experimental.pallas import tpu as pltpu


_BLOCK_B = 256
_NSTREAMS = 4
_ROWS_EUP = 192

_LN2 = 0.6931471805599453
_P5 = (
    -89.96213512131439,
    3.4989067476988125,
    -2.418999477900546,
    1.1017396261326036,
    -0.27900102387540926,
    0.029808765243435193,
)


def _fast_ln(x):
    b = jax.lax.bitcast_convert_type(x, jnp.int32)
    e = jax.lax.shift_right_logical(b, 23).astype(jnp.float32)
    m = jax.lax.bitcast_convert_type(
        jax.lax.bitwise_or(jax.lax.bitwise_and(b, 0x007FFFFF), 0x3F800000),
        jnp.float32,
    )
    r = jnp.float32(_P5[5])
    for c in (_P5[4], _P5[3], _P5[2], _P5[1], _P5[0]):
        r = r * m + jnp.float32(c)
    return r + e * jnp.float32(_LN2)


def _masked_rowtree_sum(lg, yb, row0):
    cols = jax.lax.broadcasted_iota(jnp.int32, lg.shape, 1)
    m = jnp.where(cols == yb, 0.0, lg)
    return jnp.sum(m.reshape(m.shape[0] // 8, 8, m.shape[1]), axis=0)


def _body(*refs, nsteps, scale):
    i = pl.program_id(0)
    ns = _NSTREAMS
    y_refs = refs[:ns]
    x_refs = refs[ns : 2 * ns]
    o_ref = refs[2 * ns]
    acc_ref = refs[2 * ns + 1]
    re = _ROWS_EUP

    part = None
    for y_ref, x_ref in zip(y_refs, x_refs):
        x = x_ref[...]
        yb = y_ref[...]
        p = _masked_rowtree_sum(jnp.log(x[:re]), yb[:re], 0)
        p = p + _masked_rowtree_sum(_fast_ln(x[re:]), yb[re:], re)
        part = p if part is None else part + p

    @pl.when(i == 0)
    def _():
        acc_ref[...] = jnp.zeros_like(acc_ref)

    acc_ref[...] += part

    @pl.when(i == nsteps - 1)
    def _():
        o_ref[...] = jnp.sum(acc_ref[...]).reshape(1, 1) * scale


def kernel(y_pred, y):
    B, C = y_pred.shape
    bb = _BLOCK_B
    ns = _NSTREAMS
    nsteps = B // (bb * ns)
    scale = -1.0 / ((C - 1) * B)
    y2 = y.reshape(B, 1).astype(jnp.int32)

    def x_spec(s):
        return pl.BlockSpec((bb, C), lambda i, s=s: (i + s * nsteps, 0))

    def y_spec(s):
        return pl.BlockSpec((bb, 1), lambda i, s=s: (i + s * nsteps, 0))

    out = pl.pallas_call(
        functools.partial(_body, nsteps=nsteps, scale=scale),
        grid=(nsteps,),
        in_specs=[y_spec(s) for s in range(ns)] + [x_spec(s) for s in range(ns)],
        out_specs=pl.BlockSpec((1, 1), lambda i: (0, 0)),
        out_shape=jax.ShapeDtypeStruct((1, 1), jnp.float32),
        scratch_shapes=[pltpu.VMEM((8, C), jnp.float32)],
    )(*([y2] * ns + [y_pred] * ns))
    return out[0, 0]


# speedup vs baseline: 1.0278x; 1.0278x over previous
import functools

import jax
import jax.numpy as jnp
from jax.experimental import pallas as pl
from jax.experimental.pallas import tpu as pltpu


_BLOCK_B = 256
_NSTREAMS = 4
_LN2 = 0.6931471805599453


def _body(*refs, nsteps, scale):
    i = pl.program_id(0)
    ns = _NSTREAMS
    y_refs = refs[:ns]
    x_refs = refs[ns : 2 * ns]
    o_ref = refs[2 * ns]
    acc_ref = refs[2 * ns + 1]

    part = None
    for y_ref, x_ref in zip(y_refs, x_refs):
        lg = jnp.log2(x_ref[...])
        cols = jax.lax.broadcasted_iota(jnp.int32, lg.shape, 1)
        m = jnp.where(cols == y_ref[...], 0.0, lg)
        p = jnp.sum(m.reshape(m.shape[0] // 8, 8, m.shape[1]), axis=0)
        part = p if part is None else part + p

    @pl.when(i == 0)
    def _():
        acc_ref[...] = jnp.zeros_like(acc_ref)

    acc_ref[...] += part

    @pl.when(i == nsteps - 1)
    def _():
        o_ref[...] = jnp.sum(acc_ref[...]).reshape(1, 1) * scale


def kernel(y_pred, y):
    B, C = y_pred.shape
    bb = _BLOCK_B
    ns = _NSTREAMS
    nsteps = B // (bb * ns)
    scale = -_LN2 / ((C - 1) * B)
    y2 = y.reshape(B, 1).astype(jnp.int32)

    def x_spec(s):
        return pl.BlockSpec((bb, C), lambda i, s=s: (i + s * nsteps, 0))

    def y_spec(s):
        return pl.BlockSpec((bb, 1), lambda i, s=s: (i + s * nsteps, 0))

    out = pl.pallas_call(
        functools.partial(_body, nsteps=nsteps, scale=scale),
        grid=(nsteps,),
        in_specs=[y_spec(s) for s in range(ns)] + [x_spec(s) for s in range(ns)],
        out_specs=pl.BlockSpec((1, 1), lambda i: (0, 0)),
        out_shape=jax.ShapeDtypeStruct((1, 1), jnp.float32),
        scratch_shapes=[pltpu.VMEM((8, C), jnp.float32)],
    )(*([y2] * ns + [y_pred] * ns))
    return out[0, 0]
